# baseline (speedup 1.0000x reference)
import jax
import jax.numpy as jnp
from jax import lax
from jax.experimental import pallas as pl
from jax.experimental.pallas import tpu as pltpu
from jax.experimental.pallas import tpu_sc as plsc

N_NODES = 10000
N_EDGES = 320000
N_LABEL = 100000
D_IN = 128
D_OUT = 64
EPS = 1e-5

NC = 2
NS = 16
NW = NC * NS
L = 16

CHUNK = 128
E_PAD = 323584
N_PAD = 102400
NROWS = 10240

SEG_N0, SEG_N1 = 124, 34
DEC_N0, DEC_N1 = 45, 5


def _make_seg(width, with_deg):
  assert (SEG_N0 + SEG_N1) * NS * CHUNK == E_PAD
  rows = NROWS // NS
  sub = 32
  nz = rows // sub
  mesh = plsc.VectorSubcoreMesh(core_axis_name="c", subcore_axis_name="s",
                                num_cores=NC, num_subcores=NS)

  def body(rows_hbm, src_hbm, dst_hbm, *refs):
    if with_deg:
      (agg_hbm, deg_hbm, acc_s, sidx0, didx0, rows0, sidx1, didx1, rows1,
       zbuf, deg_v, sem0, sem1) = refs
    else:
      (agg_hbm, acc_s, sidx0, didx0, rows0, sidx1, didx1, rows1,
       zbuf, sem0, sem1) = refs
    c = lax.axis_index("c")
    s = lax.axis_index("s")
    iota = lax.iota(jnp.int32, L)

    def zrow(i, carry):
      def zcol(j, carry2):
        zbuf[i, pl.ds(j * L, L)] = jnp.zeros((L,), jnp.float32)
        return carry2
      return lax.fori_loop(0, width // L, zcol, carry)
    lax.fori_loop(0, sub, zrow, 0)

    def zcopy(i, carry):
      pltpu.sync_copy(zbuf, acc_s.at[pl.ds(s * rows + i * sub, sub)])
      return carry
    lax.fori_loop(0, nz, zcopy, 0)
    if with_deg:
      def zdeg(i, carry):
        deg_v[pl.ds(i * L, L)] = jnp.zeros((L,), jnp.int32)
        return carry
      lax.fori_loop(0, NROWS // L, zdeg, 0)
    plsc.subcore_barrier()

    def load_start(off, sidx, didx, rowsv, sem):
      pltpu.sync_copy(src_hbm.at[pl.ds(off, CHUNK)], sidx)
      pltpu.sync_copy(dst_hbm.at[pl.ds(off, CHUNK)], didx)
      pltpu.async_copy(rows_hbm.at[sidx], rowsv, sem)

    def hist(didx):
      def grp(j, carry):
        d16 = didx[pl.ds(j * L, L)]
        sk, _ = plsc.sort_key_val(d16, d16)
        prev = sk.at[jnp.maximum(iota - 1, 0)].get(mode="promise_in_bounds")
        nxt = sk.at[jnp.minimum(iota + 1, L - 1)].get(
            mode="promise_in_bounds")
        is_start = (iota == 0) | (sk != prev)
        is_last = (iota == L - 1) | (sk != nxt)
        start = plsc.cummax(jnp.where(is_start, iota, 0))
        count = iota - start + 1
        old = plsc.load_gather(deg_v, [sk])
        plsc.store_scatter(deg_v, [sk], old + count, mask=is_last)
        return carry
      lax.fori_loop(0, CHUNK // L, grp, 0)

    def finish(sidx, didx, rowsv, sem):
      if with_deg:
        hist(didx)
      pltpu.make_async_copy(rows_hbm.at[sidx], rowsv, sem).wait()
      pltpu.sync_copy(rowsv, acc_s.at[didx], add=True)

    def pipeline(base, nchunk):
      load_start(base, sidx0, didx0, rows0, sem0)
      def step(k2, carry):
        off_a = base + (2 * k2) * CHUNK
        load_start(off_a + CHUNK, sidx1, didx1, rows1, sem1)
        finish(sidx0, didx0, rows0, sem0)
        load_start(off_a + 2 * CHUNK, sidx0, didx0, rows0, sem0)
        finish(sidx1, didx1, rows1, sem1)
        return carry
      lax.fori_loop(0, (nchunk - 1) // 2, step, 0)
      finish(sidx0, didx0, rows0, sem0)
      if nchunk % 2 == 0:
        load_start(base + (nchunk - 1) * CHUNK, sidx1, didx1, rows1, sem1)
        finish(sidx1, didx1, rows1, sem1)

    if SEG_N0:
      @pl.when(c == 0)
      def _():
        pipeline(s * SEG_N0 * CHUNK, SEG_N0)
    if SEG_N1:
      @pl.when(c == 1)
      def _():
        pipeline((NS * SEG_N0 + s * SEG_N1) * CHUNK, SEG_N1)

    plsc.subcore_barrier()
    pltpu.sync_copy(acc_s.at[pl.ds(s * rows, rows)],
                    agg_hbm.at[pl.ds(c * NROWS + s * rows, rows)])
    if with_deg:
      pltpu.sync_copy(deg_v, deg_hbm.at[pl.ds((c * NS + s) * NROWS, NROWS)])

  out_type = [jax.ShapeDtypeStruct((NC * NROWS, width), jnp.float32)]
  scratch = [
      pltpu.VMEM_SHARED((NROWS, width), jnp.float32),
      pltpu.VMEM((CHUNK,), jnp.int32),
      pltpu.VMEM((CHUNK,), jnp.int32),
      pltpu.VMEM((CHUNK, width), jnp.float32),
      pltpu.VMEM((CHUNK,), jnp.int32),
      pltpu.VMEM((CHUNK,), jnp.int32),
      pltpu.VMEM((CHUNK, width), jnp.float32),
      pltpu.VMEM((sub, width), jnp.float32),
  ]
  if with_deg:
    out_type.append(jax.ShapeDtypeStruct((NW * NROWS,), jnp.int32))
    scratch.append(pltpu.VMEM((NROWS,), jnp.int32))
  scratch.append(pltpu.SemaphoreType.DMA)
  scratch.append(pltpu.SemaphoreType.DMA)

  return pl.kernel(body, out_type=tuple(out_type) if with_deg else out_type[0],
                   mesh=mesh, scratch_types=scratch,
                   compiler_params=pltpu.CompilerParams(
                       needs_layout_passes=False))


def _make_decode():
  assert (DEC_N0 + DEC_N1) * NS * CHUNK == N_PAD
  mesh = plsc.VectorSubcoreMesh(core_axis_name="c", subcore_axis_name="s",
                                num_cores=NC, num_subcores=NS)

  def body(z_hbm, la_hbm, lb_hbm, out_hbm, la0, lb0, za0, zb0, la1, lb1,
           za1, zb1, part_v, sem0, sem1):
    c = lax.axis_index("c")
    s = lax.axis_index("s")
    iota = lax.iota(jnp.int32, L)

    def load_start(off, la_v, lb_v, za_v, zb_v, sem):
      pltpu.sync_copy(la_hbm.at[pl.ds(off, CHUNK)], la_v)
      pltpu.sync_copy(lb_hbm.at[pl.ds(off, CHUNK)], lb_v)
      pltpu.async_copy(z_hbm.at[la_v], za_v, sem)
      pltpu.async_copy(z_hbm.at[lb_v], zb_v, sem)

    def finish(off, la_v, lb_v, za_v, zb_v, sem):
      pltpu.make_async_copy(z_hbm.at[la_v], za_v, sem).wait()
      pltpu.make_async_copy(z_hbm.at[lb_v], zb_v, sem).wait()
      def group(g, carry):
        pk = jnp.zeros((L,), jnp.float32)
        for t in range(L):
          r = g * L + t
          acc = za_v[r, pl.ds(0, L)] * zb_v[r, pl.ds(0, L)]
          for q in range(1, D_OUT // L):
            acc = acc + za_v[r, pl.ds(q * L, L)] * zb_v[r, pl.ds(q * L, L)]
          pk = jnp.where(iota == t, jnp.sum(acc), pk)
        part_v[pl.ds(g * L, L)] = pk
        return carry
      lax.fori_loop(0, CHUNK // L, group, 0)
      pltpu.sync_copy(part_v, out_hbm.at[pl.ds(off, CHUNK)])

    def pipeline(base, nchunk):
      load_start(base, la0, lb0, za0, zb0, sem0)
      def step(k2, carry):
        off_a = base + (2 * k2) * CHUNK
        load_start(off_a + CHUNK, la1, lb1, za1, zb1, sem1)
        finish(off_a, la0, lb0, za0, zb0, sem0)
        load_start(off_a + 2 * CHUNK, la0, lb0, za0, zb0, sem0)
        finish(off_a + CHUNK, la1, lb1, za1, zb1, sem1)
        return carry
      lax.fori_loop(0, (nchunk - 1) // 2, step, 0)
      last_even = base + (nchunk - 1 if nchunk % 2 else nchunk - 2) * CHUNK
      finish(last_even, la0, lb0, za0, zb0, sem0)
      if nchunk % 2 == 0:
        off = base + (nchunk - 1) * CHUNK
        load_start(off, la1, lb1, za1, zb1, sem1)
        finish(off, la1, lb1, za1, zb1, sem1)

    if DEC_N0:
      @pl.when(c == 0)
      def _():
        pipeline(s * DEC_N0 * CHUNK, DEC_N0)
    if DEC_N1:
      @pl.when(c == 1)
      def _():
        pipeline((NS * DEC_N0 + s * DEC_N1) * CHUNK, DEC_N1)

  return pl.kernel(
      body,
      out_type=jax.ShapeDtypeStruct((N_PAD,), jnp.float32),
      mesh=mesh,
      compiler_params=pltpu.CompilerParams(needs_layout_passes=False,
                                           use_tc_tiling_on_sc=False),
      scratch_types=[
          pltpu.VMEM((CHUNK,), jnp.int32),
          pltpu.VMEM((CHUNK,), jnp.int32),
          pltpu.VMEM((CHUNK, D_OUT), jnp.float32),
          pltpu.VMEM((CHUNK, D_OUT), jnp.float32),
          pltpu.VMEM((CHUNK,), jnp.int32),
          pltpu.VMEM((CHUNK,), jnp.int32),
          pltpu.VMEM((CHUNK, D_OUT), jnp.float32),
          pltpu.VMEM((CHUNK, D_OUT), jnp.float32),
          pltpu.VMEM((CHUNK,), jnp.float32),
          pltpu.SemaphoreType.DMA,
          pltpu.SemaphoreType.DMA,
      ],
  )


def _sage_bn(x_prev, agg, deg, wl, bl, wr, g, be):
  mean = agg / jnp.maximum(deg, 1.0)
  h = (lax.dot_general(mean, wl, (((1,), (1,)), ((), ())),
                       preferred_element_type=jnp.float32)
       + bl
       + lax.dot_general(x_prev, wr, (((1,), (1,)), ((), ())),
                         preferred_element_type=jnp.float32))
  mu = jnp.mean(h, axis=0, keepdims=True)
  var = jnp.mean((h - mu) ** 2, axis=0, keepdims=True)
  hn = (h - mu) / jnp.sqrt(var + EPS) * g + be
  return jnp.maximum(hn, 0.0)


def _deg_col(dg_ref):
  deg = jnp.sum(dg_ref[...], axis=1, keepdims=True).astype(jnp.float32)
  return deg[:N_NODES, :]


def _dense1_body(x_ref, a1_ref, dg_ref, wl_ref, bl_ref, wr_ref, g_ref, be_ref,
                 out_ref):
  agg = a1_ref[0] + a1_ref[1]
  out_ref[...] = _sage_bn(x_ref[...], agg, _deg_col(dg_ref), wl_ref[...],
                          bl_ref[...], wr_ref[...], g_ref[...], be_ref[...])


def _dense2_body(h1_ref, a2_ref, dg_ref, wl_ref, bl_ref, wr_ref, g_ref,
                 be_ref, wlin_ref, blin_ref, z_ref):
  agg = a2_ref[0] + a2_ref[1]
  deg = _deg_col(dg_ref)
  h2 = _sage_bn(h1_ref[...], agg, deg, wl_ref[...], bl_ref[...], wr_ref[...],
                g_ref[...], be_ref[...])
  z_ref[...] = (lax.dot_general(h2, wlin_ref[...], (((1,), (1,)), ((), ())),
                                preferred_element_type=jnp.float32)
                + blin_ref[...])


_seg1 = _make_seg(D_IN, with_deg=True)
_seg2 = _make_seg(D_IN, with_deg=False)
_decode = _make_decode()

_dense1 = pl.pallas_call(
    _dense1_body,
    out_shape=jax.ShapeDtypeStruct((N_NODES, D_IN), jnp.float32),
)

_dense2 = pl.pallas_call(
    _dense2_body,
    out_shape=jax.ShapeDtypeStruct((N_NODES, D_OUT), jnp.float32),
)


def kernel(x, edge_index, edge_label_index, W_l1, b_l1, W_r1, g1, be1, W_l2,
           b_l2, W_r2, g2, be2, W_lin, b_lin):
  src = edge_index[0].astype(jnp.int32)
  dst = edge_index[1].astype(jnp.int32)
  la = edge_label_index[0].astype(jnp.int32)
  lb = edge_label_index[1].astype(jnp.int32)
  epad = E_PAD - src.shape[0]
  src = jnp.concatenate([src, jnp.zeros((epad,), jnp.int32)])
  dst = jnp.concatenate([dst, jnp.full((epad,), N_NODES, jnp.int32)])
  pad = N_PAD - la.shape[0]
  la = jnp.concatenate([la, jnp.zeros((pad,), jnp.int32)])
  lb = jnp.concatenate([lb, jnp.zeros((pad,), jnp.int32)])

  a1, degs = _seg1(x, src, dst)
  a1 = a1.reshape(NC, NROWS, D_IN)[:, :N_NODES]
  dg = degs.reshape(NW, NROWS).T
  h1 = _dense1(x, a1, dg, W_l1, b_l1.reshape(1, -1), W_r1, g1.reshape(1, -1),
               be1.reshape(1, -1))
  a2 = _seg2(h1, src, dst).reshape(NC, NROWS, D_IN)[:, :N_NODES]
  z = _dense2(h1, a2, dg, W_l2, b_l2.reshape(1, -1), W_r2,
              g2.reshape(1, -1), be2.reshape(1, -1), W_lin,
              b_lin.reshape(1, -1))
  return _decode(z, la, lb)[:N_LABEL]


# speedup vs baseline: 1.0122x; 1.0122x over previous
import jax
import jax.numpy as jnp
from jax import lax
from jax.experimental import pallas as pl
from jax.experimental.pallas import tpu as pltpu
from jax.experimental.pallas import tpu_sc as plsc

N_NODES = 10000
N_EDGES = 320000
N_LABEL = 100000
D_IN = 128
D_OUT = 64
EPS = 1e-5

NC = 2
NS = 16
NW = NC * NS
L = 16

CHUNK = 128
E_PAD = 323584
N_PAD = 102400
NROWS = 10240

SEG_N0, SEG_N1 = 118, 40
DEC_N0, DEC_N1 = 45, 5


def _make_seg(width, with_deg):
  assert (SEG_N0 + SEG_N1) * NS * CHUNK == E_PAD
  rows = NROWS // NS
  sub = 32
  nz = rows // sub
  mesh = plsc.VectorSubcoreMesh(core_axis_name="c", subcore_axis_name="s",
                                num_cores=NC, num_subcores=NS)

  def body(rows_hbm, src_hbm, dst_hbm, *refs):
    if with_deg:
      (agg_hbm, deg_hbm, acc_s, sidx0, didx0, rows0, sidx1, didx1, rows1,
       zbuf, deg_v, sem0, sem1) = refs
    else:
      (agg_hbm, acc_s, sidx0, didx0, rows0, sidx1, didx1, rows1,
       zbuf, sem0, sem1) = refs
    c = lax.axis_index("c")
    s = lax.axis_index("s")
    iota = lax.iota(jnp.int32, L)

    def zrow(i, carry):
      def zcol(j, carry2):
        zbuf[i, pl.ds(j * L, L)] = jnp.zeros((L,), jnp.float32)
        return carry2
      return lax.fori_loop(0, width // L, zcol, carry)
    lax.fori_loop(0, sub, zrow, 0)

    def zcopy(i, carry):
      pltpu.sync_copy(zbuf, acc_s.at[pl.ds(s * rows + i * sub, sub)])
      return carry
    lax.fori_loop(0, nz, zcopy, 0)
    if with_deg:
      def zdeg(i, carry):
        deg_v[pl.ds(i * L, L)] = jnp.zeros((L,), jnp.int32)
        return carry
      lax.fori_loop(0, NROWS // L, zdeg, 0)
    plsc.subcore_barrier()

    def load_start(off, sidx, didx, rowsv, sem):
      pltpu.sync_copy(src_hbm.at[pl.ds(off, CHUNK)], sidx)
      pltpu.sync_copy(dst_hbm.at[pl.ds(off, CHUNK)], didx)
      pltpu.async_copy(rows_hbm.at[sidx], rowsv, sem)

    def hist(didx):
      def grp(j, carry):
        d16 = didx[pl.ds(j * L, L)]
        sk, _ = plsc.sort_key_val(d16, d16)
        prev = sk.at[jnp.maximum(iota - 1, 0)].get(mode="promise_in_bounds")
        nxt = sk.at[jnp.minimum(iota + 1, L - 1)].get(
            mode="promise_in_bounds")
        is_start = (iota == 0) | (sk != prev)
        is_last = (iota == L - 1) | (sk != nxt)
        start = plsc.cummax(jnp.where(is_start, iota, 0))
        count = iota - start + 1
        old = plsc.load_gather(deg_v, [sk])
        plsc.store_scatter(deg_v, [sk], old + count, mask=is_last)
        return carry
      lax.fori_loop(0, CHUNK // L, grp, 0)

    def finish(sidx, didx, rowsv, sem):
      if with_deg:
        hist(didx)
      pltpu.make_async_copy(rows_hbm.at[sidx], rowsv, sem).wait()
      pltpu.sync_copy(rowsv, acc_s.at[didx], add=True)

    def pipeline(base, nchunk):
      load_start(base, sidx0, didx0, rows0, sem0)
      def step(k2, carry):
        off_a = base + (2 * k2) * CHUNK
        load_start(off_a + CHUNK, sidx1, didx1, rows1, sem1)
        finish(sidx0, didx0, rows0, sem0)
        load_start(off_a + 2 * CHUNK, sidx0, didx0, rows0, sem0)
        finish(sidx1, didx1, rows1, sem1)
        return carry
      lax.fori_loop(0, (nchunk - 1) // 2, step, 0)
      finish(sidx0, didx0, rows0, sem0)
      if nchunk % 2 == 0:
        load_start(base + (nchunk - 1) * CHUNK, sidx1, didx1, rows1, sem1)
        finish(sidx1, didx1, rows1, sem1)

    if SEG_N0:
      @pl.when(c == 0)
      def _():
        pipeline(s * SEG_N0 * CHUNK, SEG_N0)
    if SEG_N1:
      @pl.when(c == 1)
      def _():
        pipeline((NS * SEG_N0 + s * SEG_N1) * CHUNK, SEG_N1)

    plsc.subcore_barrier()
    pltpu.sync_copy(acc_s.at[pl.ds(s * rows, rows)],
                    agg_hbm.at[pl.ds(c * NROWS + s * rows, rows)])
    if with_deg:
      pltpu.sync_copy(deg_v, deg_hbm.at[pl.ds((c * NS + s) * NROWS, NROWS)])

  out_type = [jax.ShapeDtypeStruct((NC * NROWS, width), jnp.float32)]
  scratch = [
      pltpu.VMEM_SHARED((NROWS, width), jnp.float32),
      pltpu.VMEM((CHUNK,), jnp.int32),
      pltpu.VMEM((CHUNK,), jnp.int32),
      pltpu.VMEM((CHUNK, width), jnp.float32),
      pltpu.VMEM((CHUNK,), jnp.int32),
      pltpu.VMEM((CHUNK,), jnp.int32),
      pltpu.VMEM((CHUNK, width), jnp.float32),
      pltpu.VMEM((sub, width), jnp.float32),
  ]
  if with_deg:
    out_type.append(jax.ShapeDtypeStruct((NW * NROWS,), jnp.int32))
    scratch.append(pltpu.VMEM((NROWS,), jnp.int32))
  scratch.append(pltpu.SemaphoreType.DMA)
  scratch.append(pltpu.SemaphoreType.DMA)

  return pl.kernel(body, out_type=tuple(out_type) if with_deg else out_type[0],
                   mesh=mesh, scratch_types=scratch,
                   compiler_params=pltpu.CompilerParams(
                       needs_layout_passes=False))


def _make_decode():
  assert (DEC_N0 + DEC_N1) * NS * CHUNK == N_PAD
  mesh = plsc.VectorSubcoreMesh(core_axis_name="c", subcore_axis_name="s",
                                num_cores=NC, num_subcores=NS)

  def body(z_hbm, la_hbm, lb_hbm, out_hbm, la0, lb0, za0, zb0, la1, lb1,
           za1, zb1, part_v, sem0, sem1):
    c = lax.axis_index("c")
    s = lax.axis_index("s")
    iota = lax.iota(jnp.int32, L)

    def load_start(off, la_v, lb_v, za_v, zb_v, sem):
      pltpu.sync_copy(la_hbm.at[pl.ds(off, CHUNK)], la_v)
      pltpu.sync_copy(lb_hbm.at[pl.ds(off, CHUNK)], lb_v)
      pltpu.async_copy(z_hbm.at[la_v], za_v, sem)
      pltpu.async_copy(z_hbm.at[lb_v], zb_v, sem)

    def finish(off, la_v, lb_v, za_v, zb_v, sem):
      pltpu.make_async_copy(z_hbm.at[la_v], za_v, sem).wait()
      pltpu.make_async_copy(z_hbm.at[lb_v], zb_v, sem).wait()
      def group(g, carry):
        pk = jnp.zeros((L,), jnp.float32)
        for t in range(L):
          r = g * L + t
          acc = za_v[r, pl.ds(0, L)] * zb_v[r, pl.ds(0, L)]
          for q in range(1, D_OUT // L):
            acc = acc + za_v[r, pl.ds(q * L, L)] * zb_v[r, pl.ds(q * L, L)]
          pk = jnp.where(iota == t, jnp.sum(acc), pk)
        part_v[pl.ds(g * L, L)] = pk
        return carry
      lax.fori_loop(0, CHUNK // L, group, 0)
      pltpu.sync_copy(part_v, out_hbm.at[pl.ds(off, CHUNK)])

    def pipeline(base, nchunk):
      load_start(base, la0, lb0, za0, zb0, sem0)
      def step(k2, carry):
        off_a = base + (2 * k2) * CHUNK
        load_start(off_a + CHUNK, la1, lb1, za1, zb1, sem1)
        finish(off_a, la0, lb0, za0, zb0, sem0)
        load_start(off_a + 2 * CHUNK, la0, lb0, za0, zb0, sem0)
        finish(off_a + CHUNK, la1, lb1, za1, zb1, sem1)
        return carry
      lax.fori_loop(0, (nchunk - 1) // 2, step, 0)
      last_even = base + (nchunk - 1 if nchunk % 2 else nchunk - 2) * CHUNK
      finish(last_even, la0, lb0, za0, zb0, sem0)
      if nchunk % 2 == 0:
        off = base + (nchunk - 1) * CHUNK
        load_start(off, la1, lb1, za1, zb1, sem1)
        finish(off, la1, lb1, za1, zb1, sem1)

    if DEC_N0:
      @pl.when(c == 0)
      def _():
        pipeline(s * DEC_N0 * CHUNK, DEC_N0)
    if DEC_N1:
      @pl.when(c == 1)
      def _():
        pipeline((NS * DEC_N0 + s * DEC_N1) * CHUNK, DEC_N1)

  return pl.kernel(
      body,
      out_type=jax.ShapeDtypeStruct((N_PAD,), jnp.float32),
      mesh=mesh,
      compiler_params=pltpu.CompilerParams(needs_layout_passes=False,
                                           use_tc_tiling_on_sc=False),
      scratch_types=[
          pltpu.VMEM((CHUNK,), jnp.int32),
          pltpu.VMEM((CHUNK,), jnp.int32),
          pltpu.VMEM((CHUNK, D_OUT), jnp.float32),
          pltpu.VMEM((CHUNK, D_OUT), jnp.float32),
          pltpu.VMEM((CHUNK,), jnp.int32),
          pltpu.VMEM((CHUNK,), jnp.int32),
          pltpu.VMEM((CHUNK, D_OUT), jnp.float32),
          pltpu.VMEM((CHUNK, D_OUT), jnp.float32),
          pltpu.VMEM((CHUNK,), jnp.float32),
          pltpu.SemaphoreType.DMA,
          pltpu.SemaphoreType.DMA,
      ],
  )


def _sage_bn(x_prev, agg, deg, wl, bl, wr, g, be):
  mean = agg / jnp.maximum(deg, 1.0)
  h = (lax.dot_general(mean, wl, (((1,), (1,)), ((), ())),
                       preferred_element_type=jnp.float32)
       + bl
       + lax.dot_general(x_prev, wr, (((1,), (1,)), ((), ())),
                         preferred_element_type=jnp.float32))
  mu = jnp.mean(h, axis=0, keepdims=True)
  var = jnp.mean((h - mu) ** 2, axis=0, keepdims=True)
  hn = (h - mu) / jnp.sqrt(var + EPS) * g + be
  return jnp.maximum(hn, 0.0)


def _deg_col(dg_ref):
  deg = jnp.sum(dg_ref[...], axis=1, keepdims=True).astype(jnp.float32)
  return deg[:N_NODES, :]


def _dense1_body(x_ref, a1_ref, dg_ref, wl_ref, bl_ref, wr_ref, g_ref, be_ref,
                 out_ref):
  agg = a1_ref[0] + a1_ref[1]
  out_ref[...] = _sage_bn(x_ref[...], agg, _deg_col(dg_ref), wl_ref[...],
                          bl_ref[...], wr_ref[...], g_ref[...], be_ref[...])


def _dense2_body(h1_ref, a2_ref, dg_ref, wl_ref, bl_ref, wr_ref, g_ref,
                 be_ref, wlin_ref, blin_ref, z_ref):
  agg = a2_ref[0] + a2_ref[1]
  deg = _deg_col(dg_ref)
  h2 = _sage_bn(h1_ref[...], agg, deg, wl_ref[...], bl_ref[...], wr_ref[...],
                g_ref[...], be_ref[...])
  z_ref[...] = (lax.dot_general(h2, wlin_ref[...], (((1,), (1,)), ((), ())),
                                preferred_element_type=jnp.float32)
                + blin_ref[...])


_seg1 = _make_seg(D_IN, with_deg=True)
_seg2 = _make_seg(D_IN, with_deg=False)
_decode = _make_decode()

_dense1 = pl.pallas_call(
    _dense1_body,
    out_shape=jax.ShapeDtypeStruct((N_NODES, D_IN), jnp.float32),
)

_dense2 = pl.pallas_call(
    _dense2_body,
    out_shape=jax.ShapeDtypeStruct((N_NODES, D_OUT), jnp.float32),
)


def kernel(x, edge_index, edge_label_index, W_l1, b_l1, W_r1, g1, be1, W_l2,
           b_l2, W_r2, g2, be2, W_lin, b_lin):
  src = edge_index[0].astype(jnp.int32)
  dst = edge_index[1].astype(jnp.int32)
  la = edge_label_index[0].astype(jnp.int32)
  lb = edge_label_index[1].astype(jnp.int32)
  epad = E_PAD - src.shape[0]
  src = jnp.concatenate([src, jnp.zeros((epad,), jnp.int32)])
  dst = jnp.concatenate([dst, jnp.full((epad,), N_NODES, jnp.int32)])
  pad = N_PAD - la.shape[0]
  la = jnp.concatenate([la, jnp.zeros((pad,), jnp.int32)])
  lb = jnp.concatenate([lb, jnp.zeros((pad,), jnp.int32)])

  a1, degs = _seg1(x, src, dst)
  a1 = a1.reshape(NC, NROWS, D_IN)[:, :N_NODES]
  dg = degs.reshape(NW, NROWS).T
  h1 = _dense1(x, a1, dg, W_l1, b_l1.reshape(1, -1), W_r1, g1.reshape(1, -1),
               be1.reshape(1, -1))
  a2 = _seg2(h1, src, dst).reshape(NC, NROWS, D_IN)[:, :N_NODES]
  z = _dense2(h1, a2, dg, W_l2, b_l2.reshape(1, -1), W_r2,
              g2.reshape(1, -1), be2.reshape(1, -1), W_lin,
              b_lin.reshape(1, -1))
  return _decode(z, la, lb)[:N_LABEL]


# speedup vs baseline: 1.0134x; 1.0012x over previous
import jax
import jax.numpy as jnp
from jax import lax
from jax.experimental import pallas as pl
from jax.experimental.pallas import tpu as pltpu
from jax.experimental.pallas import tpu_sc as plsc

N_NODES = 10000
N_EDGES = 320000
N_LABEL = 100000
D_IN = 128
D_OUT = 64
EPS = 1e-5

NC = 2
NS = 16
NW = NC * NS
L = 16

CHUNK = 128
E_PAD = 323584
N_PAD = 102400
NROWS = 10240

SEG_N0, SEG_N1 = 118, 40
DEC_N0, DEC_N1 = 23, 2
DEC_CHUNK = 256


def _make_seg(width, with_deg):
  assert (SEG_N0 + SEG_N1) * NS * CHUNK == E_PAD
  rows = NROWS // NS
  sub = 32
  nz = rows // sub
  mesh = plsc.VectorSubcoreMesh(core_axis_name="c", subcore_axis_name="s",
                                num_cores=NC, num_subcores=NS)

  def body(rows_hbm, src_hbm, dst_hbm, *refs):
    if with_deg:
      (agg_hbm, deg_hbm, acc_s, sidx0, didx0, rows0, sidx1, didx1, rows1,
       zbuf, deg_v, sem0, sem1) = refs
    else:
      (agg_hbm, acc_s, sidx0, didx0, rows0, sidx1, didx1, rows1,
       zbuf, sem0, sem1) = refs
    c = lax.axis_index("c")
    s = lax.axis_index("s")
    iota = lax.iota(jnp.int32, L)

    def zrow(i, carry):
      def zcol(j, carry2):
        zbuf[i, pl.ds(j * L, L)] = jnp.zeros((L,), jnp.float32)
        return carry2
      return lax.fori_loop(0, width // L, zcol, carry)
    lax.fori_loop(0, sub, zrow, 0)

    def zcopy(i, carry):
      pltpu.sync_copy(zbuf, acc_s.at[pl.ds(s * rows + i * sub, sub)])
      return carry
    lax.fori_loop(0, nz, zcopy, 0)
    if with_deg:
      def zdeg(i, carry):
        deg_v[pl.ds(i * L, L)] = jnp.zeros((L,), jnp.int32)
        return carry
      lax.fori_loop(0, NROWS // L, zdeg, 0)
    plsc.subcore_barrier()

    def load_start(off, sidx, didx, rowsv, sem):
      pltpu.sync_copy(src_hbm.at[pl.ds(off, CHUNK)], sidx)
      pltpu.sync_copy(dst_hbm.at[pl.ds(off, CHUNK)], didx)
      pltpu.async_copy(rows_hbm.at[sidx], rowsv, sem)

    def hist(didx):
      def grp(j, carry):
        d16 = didx[pl.ds(j * L, L)]
        sk, _ = plsc.sort_key_val(d16, d16)
        prev = sk.at[jnp.maximum(iota - 1, 0)].get(mode="promise_in_bounds")
        nxt = sk.at[jnp.minimum(iota + 1, L - 1)].get(
            mode="promise_in_bounds")
        is_start = (iota == 0) | (sk != prev)
        is_last = (iota == L - 1) | (sk != nxt)
        start = plsc.cummax(jnp.where(is_start, iota, 0))
        count = iota - start + 1
        old = plsc.load_gather(deg_v, [sk])
        plsc.store_scatter(deg_v, [sk], old + count, mask=is_last)
        return carry
      lax.fori_loop(0, CHUNK // L, grp, 0)

    def finish(sidx, didx, rowsv, sem):
      if with_deg:
        hist(didx)
      pltpu.make_async_copy(rows_hbm.at[sidx], rowsv, sem).wait()
      pltpu.sync_copy(rowsv, acc_s.at[didx], add=True)

    def pipeline(base, nchunk):
      load_start(base, sidx0, didx0, rows0, sem0)
      def step(k2, carry):
        off_a = base + (2 * k2) * CHUNK
        load_start(off_a + CHUNK, sidx1, didx1, rows1, sem1)
        finish(sidx0, didx0, rows0, sem0)
        load_start(off_a + 2 * CHUNK, sidx0, didx0, rows0, sem0)
        finish(sidx1, didx1, rows1, sem1)
        return carry
      lax.fori_loop(0, (nchunk - 1) // 2, step, 0)
      finish(sidx0, didx0, rows0, sem0)
      if nchunk % 2 == 0:
        load_start(base + (nchunk - 1) * CHUNK, sidx1, didx1, rows1, sem1)
        finish(sidx1, didx1, rows1, sem1)

    if SEG_N0:
      @pl.when(c == 0)
      def _():
        pipeline(s * SEG_N0 * CHUNK, SEG_N0)
    if SEG_N1:
      @pl.when(c == 1)
      def _():
        pipeline((NS * SEG_N0 + s * SEG_N1) * CHUNK, SEG_N1)

    plsc.subcore_barrier()
    pltpu.sync_copy(acc_s.at[pl.ds(s * rows, rows)],
                    agg_hbm.at[pl.ds(c * NROWS + s * rows, rows)])
    if with_deg:
      pltpu.sync_copy(deg_v, deg_hbm.at[pl.ds((c * NS + s) * NROWS, NROWS)])

  out_type = [jax.ShapeDtypeStruct((NC * NROWS, width), jnp.float32)]
  scratch = [
      pltpu.VMEM_SHARED((NROWS, width), jnp.float32),
      pltpu.VMEM((CHUNK,), jnp.int32),
      pltpu.VMEM((CHUNK,), jnp.int32),
      pltpu.VMEM((CHUNK, width), jnp.float32),
      pltpu.VMEM((CHUNK,), jnp.int32),
      pltpu.VMEM((CHUNK,), jnp.int32),
      pltpu.VMEM((CHUNK, width), jnp.float32),
      pltpu.VMEM((sub, width), jnp.float32),
  ]
  if with_deg:
    out_type.append(jax.ShapeDtypeStruct((NW * NROWS,), jnp.int32))
    scratch.append(pltpu.VMEM((NROWS,), jnp.int32))
  scratch.append(pltpu.SemaphoreType.DMA)
  scratch.append(pltpu.SemaphoreType.DMA)

  return pl.kernel(body, out_type=tuple(out_type) if with_deg else out_type[0],
                   mesh=mesh, scratch_types=scratch,
                   compiler_params=pltpu.CompilerParams(
                       needs_layout_passes=False))


def _make_decode():
  assert (DEC_N0 + DEC_N1) * NS * DEC_CHUNK == N_PAD
  mesh = plsc.VectorSubcoreMesh(core_axis_name="c", subcore_axis_name="s",
                                num_cores=NC, num_subcores=NS)

  def body(z_hbm, la_hbm, lb_hbm, out_hbm, la0, lb0, za0, zb0, la1, lb1,
           za1, zb1, part_v, sem0, sem1):
    c = lax.axis_index("c")
    s = lax.axis_index("s")
    iota = lax.iota(jnp.int32, L)

    def load_start(off, la_v, lb_v, za_v, zb_v, sem):
      pltpu.sync_copy(la_hbm.at[pl.ds(off, DEC_CHUNK)], la_v)
      pltpu.sync_copy(lb_hbm.at[pl.ds(off, DEC_CHUNK)], lb_v)
      pltpu.async_copy(z_hbm.at[la_v], za_v, sem)
      pltpu.async_copy(z_hbm.at[lb_v], zb_v, sem)

    def finish(off, la_v, lb_v, za_v, zb_v, sem):
      pltpu.make_async_copy(z_hbm.at[la_v], za_v, sem).wait()
      pltpu.make_async_copy(z_hbm.at[lb_v], zb_v, sem).wait()
      def group(g, carry):
        pk = jnp.zeros((L,), jnp.float32)
        for t in range(L):
          r = g * L + t
          acc = za_v[r, pl.ds(0, L)] * zb_v[r, pl.ds(0, L)]
          for q in range(1, D_OUT // L):
            acc = acc + za_v[r, pl.ds(q * L, L)] * zb_v[r, pl.ds(q * L, L)]
          pk = jnp.where(iota == t, jnp.sum(acc), pk)
        part_v[pl.ds(g * L, L)] = pk
        return carry
      lax.fori_loop(0, DEC_CHUNK // L, group, 0)
      pltpu.sync_copy(part_v, out_hbm.at[pl.ds(off, DEC_CHUNK)])

    def pipeline(base, nchunk):
      load_start(base, la0, lb0, za0, zb0, sem0)
      def step(k2, carry):
        off_a = base + (2 * k2) * DEC_CHUNK
        load_start(off_a + DEC_CHUNK, la1, lb1, za1, zb1, sem1)
        finish(off_a, la0, lb0, za0, zb0, sem0)
        load_start(off_a + 2 * DEC_CHUNK, la0, lb0, za0, zb0, sem0)
        finish(off_a + DEC_CHUNK, la1, lb1, za1, zb1, sem1)
        return carry
      lax.fori_loop(0, (nchunk - 1) // 2, step, 0)
      last_even = base + (nchunk - 1 if nchunk % 2 else nchunk - 2) * DEC_CHUNK
      finish(last_even, la0, lb0, za0, zb0, sem0)
      if nchunk % 2 == 0:
        off = base + (nchunk - 1) * DEC_CHUNK
        load_start(off, la1, lb1, za1, zb1, sem1)
        finish(off, la1, lb1, za1, zb1, sem1)

    if DEC_N0:
      @pl.when(c == 0)
      def _():
        pipeline(s * DEC_N0 * DEC_CHUNK, DEC_N0)
    if DEC_N1:
      @pl.when(c == 1)
      def _():
        pipeline((NS * DEC_N0 + s * DEC_N1) * DEC_CHUNK, DEC_N1)

  return pl.kernel(
      body,
      out_type=jax.ShapeDtypeStruct((N_PAD,), jnp.float32),
      mesh=mesh,
      compiler_params=pltpu.CompilerParams(needs_layout_passes=False,
                                           use_tc_tiling_on_sc=False),
      scratch_types=[
          pltpu.VMEM((DEC_CHUNK,), jnp.int32),
          pltpu.VMEM((DEC_CHUNK,), jnp.int32),
          pltpu.VMEM((DEC_CHUNK, D_OUT), jnp.float32),
          pltpu.VMEM((DEC_CHUNK, D_OUT), jnp.float32),
          pltpu.VMEM((DEC_CHUNK,), jnp.int32),
          pltpu.VMEM((DEC_CHUNK,), jnp.int32),
          pltpu.VMEM((DEC_CHUNK, D_OUT), jnp.float32),
          pltpu.VMEM((DEC_CHUNK, D_OUT), jnp.float32),
          pltpu.VMEM((DEC_CHUNK,), jnp.float32),
          pltpu.SemaphoreType.DMA,
          pltpu.SemaphoreType.DMA,
      ],
  )


def _sage_bn(x_prev, agg, deg, wl, bl, wr, g, be):
  mean = agg / jnp.maximum(deg, 1.0)
  h = (lax.dot_general(mean, wl, (((1,), (1,)), ((), ())),
                       preferred_element_type=jnp.float32)
       + bl
       + lax.dot_general(x_prev, wr, (((1,), (1,)), ((), ())),
                         preferred_element_type=jnp.float32))
  mu = jnp.mean(h, axis=0, keepdims=True)
  var = jnp.mean((h - mu) ** 2, axis=0, keepdims=True)
  hn = (h - mu) / jnp.sqrt(var + EPS) * g + be
  return jnp.maximum(hn, 0.0)


def _deg_col(dg_ref):
  deg = jnp.sum(dg_ref[...], axis=1, keepdims=True).astype(jnp.float32)
  return deg[:N_NODES, :]


def _dense1_body(x_ref, a1_ref, dg_ref, wl_ref, bl_ref, wr_ref, g_ref, be_ref,
                 out_ref):
  agg = a1_ref[0] + a1_ref[1]
  out_ref[...] = _sage_bn(x_ref[...], agg, _deg_col(dg_ref), wl_ref[...],
                          bl_ref[...], wr_ref[...], g_ref[...], be_ref[...])


def _dense2_body(h1_ref, a2_ref, dg_ref, wl_ref, bl_ref, wr_ref, g_ref,
                 be_ref, wlin_ref, blin_ref, z_ref):
  agg = a2_ref[0] + a2_ref[1]
  deg = _deg_col(dg_ref)
  h2 = _sage_bn(h1_ref[...], agg, deg, wl_ref[...], bl_ref[...], wr_ref[...],
                g_ref[...], be_ref[...])
  z_ref[...] = (lax.dot_general(h2, wlin_ref[...], (((1,), (1,)), ((), ())),
                                preferred_element_type=jnp.float32)
                + blin_ref[...])


_seg1 = _make_seg(D_IN, with_deg=True)
_seg2 = _make_seg(D_IN, with_deg=False)
_decode = _make_decode()

_dense1 = pl.pallas_call(
    _dense1_body,
    out_shape=jax.ShapeDtypeStruct((N_NODES, D_IN), jnp.float32),
)

_dense2 = pl.pallas_call(
    _dense2_body,
    out_shape=jax.ShapeDtypeStruct((N_NODES, D_OUT), jnp.float32),
)


def kernel(x, edge_index, edge_label_index, W_l1, b_l1, W_r1, g1, be1, W_l2,
           b_l2, W_r2, g2, be2, W_lin, b_lin):
  src = edge_index[0].astype(jnp.int32)
  dst = edge_index[1].astype(jnp.int32)
  la = edge_label_index[0].astype(jnp.int32)
  lb = edge_label_index[1].astype(jnp.int32)
  epad = E_PAD - src.shape[0]
  src = jnp.concatenate([src, jnp.zeros((epad,), jnp.int32)])
  dst = jnp.concatenate([dst, jnp.full((epad,), N_NODES, jnp.int32)])
  pad = N_PAD - la.shape[0]
  la = jnp.concatenate([la, jnp.zeros((pad,), jnp.int32)])
  lb = jnp.concatenate([lb, jnp.zeros((pad,), jnp.int32)])

  a1, degs = _seg1(x, src, dst)
  a1 = a1.reshape(NC, NROWS, D_IN)[:, :N_NODES]
  dg = degs.reshape(NW, NROWS).T
  h1 = _dense1(x, a1, dg, W_l1, b_l1.reshape(1, -1), W_r1, g1.reshape(1, -1),
               be1.reshape(1, -1))
  a2 = _seg2(h1, src, dst).reshape(NC, NROWS, D_IN)[:, :N_NODES]
  z = _dense2(h1, a2, dg, W_l2, b_l2.reshape(1, -1), W_r2,
              g2.reshape(1, -1), be2.reshape(1, -1), W_lin,
              b_lin.reshape(1, -1))
  return _decode(z, la, lb)[:N_LABEL]
